# Initial kernel scaffold; baseline (speedup 1.0000x reference)
import jax
import jax.numpy as jnp
from jax.experimental import pallas as pl


def kernel(num_feature, edge_index, conv1_W, conv1_b, convA_W, convA_b, r2p_W, r2p_b, anchor_num_feature, relanchor_edge_index, num_rel, chosenrelnum):
    raise NotImplementedError("write your pallas kernel here")


# speedup vs baseline: 39.8225x; 39.8225x over previous
import jax
import jax.numpy as jnp
from jax import lax
from jax.experimental import pallas as pl
from jax.experimental.pallas import tpu as pltpu
from jax.experimental.pallas import tpu_sc as plsc

N = 10000
E = 640000
D = 128
NPAD = 10240
NW = 32
EPW = E // NW
CHUNK = 128
NCHUNK = E // CHUNK
RPT = 624


def _sc_mesh():
    return plsc.VectorSubcoreMesh(core_axis_name="c", subcore_axis_name="s")


def _hist_body(dst_hbm, histp_hbm, idxbuf, hist, redbuf, obuf, slab):
    cid = lax.axis_index("c")
    sid = lax.axis_index("s")
    wid = sid * 2 + cid

    @pl.loop(0, NPAD // 16)
    def _zero(i):
        hist[pl.ds(i * 16, 16)] = jnp.zeros((16,), jnp.float32)

    ones = jnp.ones((16,), jnp.float32)
    base = wid * EPW

    @pl.loop(0, 10)
    def _outer(jc):
        pltpu.sync_copy(dst_hbm.at[pl.ds(base + jc * 2000, 2000)], idxbuf)

        @pl.loop(0, 125)
        def _inner(i):
            idx16 = idxbuf[pl.ds(i * 16, 16)]
            plsc.addupdate_scatter(hist, [idx16], ones)

    pltpu.sync_copy(hist, slab.at[pl.ds(sid * NPAD, NPAD)])
    plsc.subcore_barrier()
    for i in range(16):
        pltpu.sync_copy(slab.at[pl.ds(i * NPAD + sid * 640, 640)], redbuf.at[i])

    @pl.loop(0, 40)
    def _red(j):
        acc = jnp.zeros((16,), jnp.float32)
        for i in range(16):
            acc = acc + redbuf[i, pl.ds(j * 16, 16)]
        obuf[pl.ds(j * 16, 16)] = acc

    pltpu.sync_copy(obuf, histp_hbm.at[pl.ds(cid * NPAD + sid * 640, 640)])


def _sc_hist(dst):
    k = pl.kernel(
        _hist_body,
        out_type=jax.ShapeDtypeStruct((2 * NPAD,), jnp.float32),
        mesh=_sc_mesh(),
        compiler_params=pltpu.CompilerParams(needs_layout_passes=False),
        scratch_types=[
            pltpu.VMEM((2000,), jnp.int32),
            pltpu.VMEM((NPAD,), jnp.float32),
            pltpu.VMEM((16, 640), jnp.float32),
            pltpu.VMEM((640,), jnp.float32),
            pltpu.VMEM_SHARED((16 * NPAD,), jnp.float32),
        ],
    )
    return k(dst)


def _scat_body(src_hbm, dst_hbm, hs_hbm, out_hbm,
               sidx0, sidx1, sidx2, didx0, didx1, didx2, sidxt, didxt,
               rows0, rows1, rows2, oshared,
               semi0, semi1, semi2, semr0, semr1, semr2,
               sems0, sems1, sems2):
    cid = lax.axis_index("c")
    sid = lax.axis_index("s")
    wid = sid * 2 + cid
    base = wid * EPW
    sidx = (sidx0, sidx1, sidx2)
    didx = (didx0, didx1, didx2)
    rows = (rows0, rows1, rows2)
    semi = (semi0, semi1, semi2)
    semr = (semr0, semr1, semr2)
    sems = (sems0, sems1, sems2)

    @pl.loop(0, 128)
    def _z(i):
        for j in range(8):
            rows0[i, pl.ds(j * 16, 16)] = jnp.zeros((16,), jnp.float32)

    rbase = sid * RPT
    for k in range(4):
        pltpu.sync_copy(rows0, oshared.at[pl.ds(rbase + k * 128, 128)])
    pltpu.sync_copy(rows0.at[pl.ds(0, 112)],
                    oshared.at[pl.ds(rbase + 512, 112)])

    @pl.when(sid == 15)
    def _ztail():
        pltpu.sync_copy(rows0.at[pl.ds(0, 16)],
                        oshared.at[pl.ds(16 * RPT, 16)])

    plsc.subcore_barrier()

    def _i_start(t, k):
        off = base + t * CHUNK
        pltpu.async_copy(src_hbm.at[pl.ds(off, CHUNK)], sidx[k], semi[k])
        pltpu.async_copy(dst_hbm.at[pl.ds(off, CHUNK)], didx[k], semi[k])

    def _i_wait(k):
        pltpu.make_async_copy(src_hbm.at[pl.ds(0, CHUNK)], sidx[k],
                              semi[k]).wait()
        pltpu.make_async_copy(dst_hbm.at[pl.ds(0, CHUNK)], didx[k],
                              semi[k]).wait()

    def _g_start(k):
        pltpu.async_copy(hs_hbm.at[sidx[k]], rows[k], semr[k])

    def _g_wait(k):
        pltpu.make_async_copy(hs_hbm.at[sidx[k]], rows[k], semr[k]).wait()

    def _s_start(k):
        pltpu.async_copy(rows[k], oshared.at[didx[k]], sems[k], add=True)

    def _s_wait(k):
        pltpu.make_async_copy(rows[k], oshared.at[didx[k]], sems[k]).wait()

    for k in range(3):
        _i_start(k, k)
    for k in range(3):
        _i_wait(k)
        _g_start(k)

    @pl.loop(0, 52)
    def _body(j):
        u = 3 * j
        for k in range(3):
            _g_wait(k)
            _s_start(k)
        for k in range(3):
            _s_wait(k)

            @pl.when(j < 51)
            def _():
                _i_start(u + 3 + k, k)
        for k in range(3):
            @pl.when(j < 51)
            def _():
                _i_wait(k)
                _g_start(k)

    toff = base + 156 * CHUNK
    pltpu.sync_copy(src_hbm.at[pl.ds(toff, 32)], sidxt)
    pltpu.sync_copy(dst_hbm.at[pl.ds(toff, 32)], didxt)
    pltpu.async_copy(hs_hbm.at[sidxt], rows0.at[pl.ds(0, 32)], semr0)
    pltpu.make_async_copy(hs_hbm.at[sidxt], rows0.at[pl.ds(0, 32)],
                          semr0).wait()
    pltpu.sync_copy(rows0.at[pl.ds(0, 32)], oshared.at[didxt], add=True)

    plsc.subcore_barrier()
    pltpu.sync_copy(oshared.at[pl.ds(rbase, RPT)],
                    out_hbm.at[cid, pl.ds(rbase, RPT)])

    @pl.when(sid == 15)
    def _wtail():
        pltpu.sync_copy(oshared.at[pl.ds(16 * RPT, 16)],
                        out_hbm.at[cid, pl.ds(16 * RPT, 16)])


def _sc_scatter(src, dst, hs):
    k = pl.kernel(
        _scat_body,
        out_type=jax.ShapeDtypeStruct((2, N, D), jnp.float32),
        mesh=_sc_mesh(),
        scratch_types=[
            pltpu.VMEM((CHUNK,), jnp.int32),
            pltpu.VMEM((CHUNK,), jnp.int32),
            pltpu.VMEM((CHUNK,), jnp.int32),
            pltpu.VMEM((CHUNK,), jnp.int32),
            pltpu.VMEM((CHUNK,), jnp.int32),
            pltpu.VMEM((CHUNK,), jnp.int32),
            pltpu.VMEM((32,), jnp.int32),
            pltpu.VMEM((32,), jnp.int32),
            pltpu.VMEM((CHUNK, D), jnp.float32),
            pltpu.VMEM((CHUNK, D), jnp.float32),
            pltpu.VMEM((CHUNK, D), jnp.float32),
            pltpu.VMEM_SHARED((N, D), jnp.float32),
            pltpu.SemaphoreType.DMA,
            pltpu.SemaphoreType.DMA,
            pltpu.SemaphoreType.DMA,
            pltpu.SemaphoreType.DMA,
            pltpu.SemaphoreType.DMA,
            pltpu.SemaphoreType.DMA,
            pltpu.SemaphoreType.DMA,
            pltpu.SemaphoreType.DMA,
            pltpu.SemaphoreType.DMA,
        ],
    )
    return k(src, dst, hs)


def _dinv_body(histp_ref, o_ref):
    p = histp_ref[...]
    o_ref[...] = lax.rsqrt(p[0] + p[1] + 1.0)[None, :]


def _tc_dinv(histp):
    return pl.pallas_call(
        _dinv_body,
        out_shape=jax.ShapeDtypeStruct((1, NPAD), jnp.float32),
    )(histp)


def _mm_body(x_ref, w_ref, dinv_ref, o_ref):
    h = lax.dot_general(x_ref[...], w_ref[...], (((1,), (1,)), ((), ())),
                        preferred_element_type=jnp.float32)
    o_ref[...] = h * dinv_ref[...]


def _tc_matmul(x, w, dinv_col):
    mb = 1000
    grid = N // mb
    return pl.pallas_call(
        _mm_body,
        grid=(grid,),
        in_specs=[
            pl.BlockSpec((mb, 1000), lambda i: (i, 0)),
            pl.BlockSpec((D, 1000), lambda i: (0, 0)),
            pl.BlockSpec((mb, 1), lambda i: (i, 0)),
        ],
        out_specs=pl.BlockSpec((mb, D), lambda i: (i, 0)),
        out_shape=jax.ShapeDtypeStruct((N, D), jnp.float32),
    )(x, w, dinv_col)


def _epi_body(p0_ref, p1_ref, hs_ref, dinv_ref, b_ref, o_ref):
    t = p0_ref[...] + p1_ref[...] + hs_ref[...]
    o_ref[...] = jnp.maximum(t * dinv_ref[...] + b_ref[...], 0.0)


def _tc_epilogue(p0, p1, hs, dinv_col, b_row):
    mb = 1000
    grid = N // mb
    return pl.pallas_call(
        _epi_body,
        grid=(grid,),
        in_specs=[
            pl.BlockSpec((mb, D), lambda i: (i, 0)),
            pl.BlockSpec((mb, D), lambda i: (i, 0)),
            pl.BlockSpec((mb, D), lambda i: (i, 0)),
            pl.BlockSpec((mb, 1), lambda i: (i, 0)),
            pl.BlockSpec((1, D), lambda i: (0, 0)),
        ],
        out_specs=pl.BlockSpec((mb, D), lambda i: (i, 0)),
        out_shape=jax.ShapeDtypeStruct((N, D), jnp.float32),
    )(p0, p1, hs, dinv_col, b_row)


def kernel(num_feature, edge_index, conv1_W, conv1_b, convA_W, convA_b,
           r2p_W, r2p_b, anchor_num_feature, relanchor_edge_index,
           num_rel, chosenrelnum):
    src = edge_index[0]
    dst = edge_index[1]
    histp = _sc_hist(dst).reshape(2, NPAD)
    dinv = _tc_dinv(histp)
    dinv_col = dinv.reshape(NPAD)[:N].reshape(N, 1)
    hs = _tc_matmul(num_feature, conv1_W, dinv_col)
    parts = _sc_scatter(src, dst, hs)
    return _tc_epilogue(parts[0], parts[1], hs, dinv_col,
                        conv1_b.reshape(1, D))
